# baseline (speedup 1.0000x reference)
import functools

import jax
import jax.numpy as jnp
from jax import lax
from jax.experimental import pallas as pl
from jax.experimental.pallas import tpu as pltpu
from jax.experimental.pallas import tpu_sc as plsc

try:
    _INFO = plsc.get_sparse_core_info()
    _NC = _INFO.num_cores
    _NS = _INFO.num_subcores
    _LANES = _INFO.num_lanes
except Exception:
    _NC, _NS, _LANES = 2, 16, 16
_NW = _NC * _NS

_BATCH = 16384
_EMBED = 16
_BPW = _BATCH // _NW
_ICH = 128
_NCHUNK = _BPW // _ICH
_OCH = _BPW // _LANES


def _body(uidx, midx, gidx, ut, mt, gt, out,
          idx_u, idx_m, idx_g, rows_u, rows_m, rows_g, out_v, sem):
    wid = lax.axis_index("s") * _NC + lax.axis_index("c")

    pltpu.sync_copy(uidx.at[wid], idx_u)
    pltpu.sync_copy(midx.at[wid], idx_m)
    pltpu.sync_copy(gidx.at[wid], idx_g)

    copies = []
    for tbl, idxr, rowsr in ((ut, idx_u, rows_u),
                             (mt, idx_m, rows_m),
                             (gt, idx_g, rows_g)):
        for j in range(_NCHUNK):
            copies.append(
                pltpu.async_copy(tbl.at[idxr.at[j]],
                                 rowsr.at[pl.ds(j * _ICH, _ICH)], sem))
    for c in copies:
        c.wait()

    lane = lax.iota(jnp.int32, _LANES)

    def chunk(c, carry):
        rows = c * _LANES + lane
        acc = jnp.zeros((_LANES,), jnp.float32)
        for d in range(_EMBED):
            col = jnp.full((_LANES,), d, jnp.int32)
            u = plsc.load_gather(rows_u, [rows, col])
            m = plsc.load_gather(rows_m, [rows, col])
            g = plsc.load_gather(rows_g, [rows, col])
            acc = acc + u * (m + g)
        out_v[c, :] = acc
        return carry

    lax.fori_loop(0, _OCH, chunk, 0)

    pltpu.sync_copy(out_v, out.at[wid])


@functools.partial(jax.jit, static_argnums=())
def _run(ui, mi, gi, ut, mt, gt):
    mesh = plsc.VectorSubcoreMesh(core_axis_name="c", subcore_axis_name="s",
                                  num_cores=_NC, num_subcores=_NS)
    fn = functools.partial(
        pl.kernel,
        mesh=mesh,
        compiler_params=pltpu.CompilerParams(
            needs_layout_passes=False, use_tc_tiling_on_sc=False),
        out_type=jax.ShapeDtypeStruct((_NW, _OCH, _LANES), jnp.float32),
        scratch_types=[
            pltpu.VMEM((_NCHUNK, _ICH), jnp.int32),
            pltpu.VMEM((_NCHUNK, _ICH), jnp.int32),
            pltpu.VMEM((_NCHUNK, _ICH), jnp.int32),
            pltpu.VMEM((_BPW, _EMBED), jnp.float32),
            pltpu.VMEM((_BPW, _EMBED), jnp.float32),
            pltpu.VMEM((_BPW, _EMBED), jnp.float32),
            pltpu.VMEM((_OCH, _LANES), jnp.float32),
            pltpu.SemaphoreType.DMA,
        ],
    )(_body)
    return fn(ui, mi, gi, ut, mt, gt)


def kernel(user_indices, movie_indices, genre_indices,
           user_table, movie_table, genre_table):
    ui = user_indices.astype(jnp.int32).reshape(_NW, _NCHUNK, _ICH)
    mi = movie_indices.astype(jnp.int32).reshape(_NW, _NCHUNK, _ICH)
    gi = genre_indices.astype(jnp.int32).reshape(_NW, _NCHUNK, _ICH)
    out = _run(ui, mi, gi, user_table, movie_table, genre_table)
    return out.reshape(_BATCH)
